# baseline (speedup 1.0000x reference)
import jax
import jax.numpy as jnp
from jax import lax
from jax.experimental import pallas as pl
from jax.experimental.pallas import tpu as pltpu
from jax.experimental.pallas import tpu_sc as plsc

_N = 10000
_E = 320000
_EMB = 128
_HID = 128
_DEPTH = 3
_LAYERS = 2

_NC = 2
_NS = 16
_NW = _NC * _NS
_EPW = _E // _NW
_CH = 40
_NCHUNK = 250
_CHALF = _NCHUNK // 2
_NPAD = 10240
_RPT = _NPAD // _NS

_BLK = 1000


def _sc_segment_sum(h, src_w, dst_w, zeros):
    mesh = plsc.VectorSubcoreMesh(core_axis_name="c", subcore_axis_name="s")

    def body(h_hbm, src_hbm, dst_hbm, zero_hbm, out_hbm,
             src_v, dst_v, rows_v, m_sh, sem):
        c = lax.axis_index("c")
        s = lax.axis_index("s")
        wid = c * _NS + s
        pltpu.sync_copy(zero_hbm.at[pl.ds(s * _RPT, _RPT)],
                        m_sh.at[pl.ds(s * _RPT, _RPT)])
        plsc.subcore_barrier()

        def chunk(j, carry):
            pltpu.async_copy(h_hbm.at[src_v.at[j]], rows_v, sem).wait()
            pltpu.sync_copy(rows_v, m_sh.at[dst_v.at[j]], add=True)
            return carry

        for half in range(2):
            pltpu.sync_copy(src_hbm.at[wid, half], src_v)
            pltpu.sync_copy(dst_hbm.at[wid, half], dst_v)
            lax.fori_loop(0, _CHALF, chunk, 0)
        plsc.subcore_barrier()
        pltpu.sync_copy(m_sh.at[pl.ds(s * _RPT, _RPT)],
                        out_hbm.at[c, pl.ds(s * _RPT, _RPT)])

    f = pl.kernel(
        body,
        out_type=jax.ShapeDtypeStruct((_NC, _NPAD, _HID), jnp.float32),
        mesh=mesh,
        scratch_types=[
            pltpu.VMEM((_CHALF, _CH), jnp.int32),
            pltpu.VMEM((_CHALF, _CH), jnp.int32),
            pltpu.VMEM((_CH, _HID), jnp.float32),
            pltpu.VMEM_SHARED((_NPAD, _HID), jnp.float32),
            pltpu.SemaphoreType.DMA,
        ],
    )
    return f(h, src_w, dst_w, zeros)


def _tc_proj(z, W_proj, b_proj):
    def body(z_ref, w_ref, b_ref, o_ref):
        o_ref[...] = (jnp.dot(z_ref[...], w_ref[...],
                              preferred_element_type=jnp.float32)
                      + b_ref[...])

    return pl.pallas_call(
        body,
        grid=(_N // _BLK,),
        in_specs=[
            pl.BlockSpec((_BLK, _EMB), lambda i: (i, 0)),
            pl.BlockSpec((_EMB, _HID), lambda i: (0, 0)),
            pl.BlockSpec((1, _HID), lambda i: (0, 0)),
        ],
        out_specs=pl.BlockSpec((_BLK, _HID), lambda i: (i, 0)),
        out_shape=jax.ShapeDtypeStruct((_N, _HID), jnp.float32),
    )(z, W_proj, b_proj.reshape(1, _HID))


def _tc_gru(m2, h, Wi, Wh, bi, bh, head=None):
    def body(m_ref, h_ref, wi_ref, wh_ref, bi_ref, bh_ref, *rest):
        inp = m_ref[0] + m_ref[1]
        hcur = h_ref[...]
        for l in range(_LAYERS):
            gi = (jnp.dot(inp, wi_ref[l], preferred_element_type=jnp.float32)
                  + bi_ref[l])
            gh = (jnp.dot(hcur, wh_ref[l], preferred_element_type=jnp.float32)
                  + bh_ref[l])
            r = jax.nn.sigmoid(gi[:, :_HID] + gh[:, :_HID])
            zg = jax.nn.sigmoid(gi[:, _HID:2 * _HID] + gh[:, _HID:2 * _HID])
            n = jnp.tanh(gi[:, 2 * _HID:] + r * gh[:, 2 * _HID:])
            hcur = (1.0 - zg) * n + zg * hcur
            inp = hcur
        if head is None:
            rest[-1][...] = hcur
        else:
            wo_ref, bo_ref, o_ref = rest
            o_ref[...] = (jnp.dot(jnp.tanh(hcur), wo_ref[...],
                                  preferred_element_type=jnp.float32)
                          + bo_ref[...])

    in_specs = [
        pl.BlockSpec((_NC, _BLK, _HID), lambda i: (0, i, 0)),
        pl.BlockSpec((_BLK, _HID), lambda i: (i, 0)),
        pl.BlockSpec((_LAYERS, _HID, 3 * _HID), lambda i: (0, 0, 0)),
        pl.BlockSpec((_LAYERS, _HID, 3 * _HID), lambda i: (0, 0, 0)),
        pl.BlockSpec((_LAYERS, 3 * _HID), lambda i: (0, 0)),
        pl.BlockSpec((_LAYERS, 3 * _HID), lambda i: (0, 0)),
    ]
    args = [m2, h, Wi, Wh, bi, bh]
    if head is None:
        out_specs = pl.BlockSpec((_BLK, _HID), lambda i: (i, 0))
        out_shape = jax.ShapeDtypeStruct((_N, _HID), jnp.float32)
    else:
        W_out, b_out = head
        in_specs += [
            pl.BlockSpec((_HID, 1), lambda i: (0, 0)),
            pl.BlockSpec((1, 1), lambda i: (0, 0)),
        ]
        args += [W_out, b_out.reshape(1, 1)]
        out_specs = pl.BlockSpec((_BLK, 1), lambda i: (i, 0))
        out_shape = jax.ShapeDtypeStruct((_N, 1), jnp.float32)

    return pl.pallas_call(
        body,
        grid=(_N // _BLK,),
        in_specs=in_specs,
        out_specs=out_specs,
        out_shape=out_shape,
    )(*args)


def kernel(z, edge_index, W_proj, b_proj, Wi, Wh, bi, bh, W_out, b_out):
    src_w = edge_index[0].reshape(_NW, 2, _CHALF, _CH)
    dst_w = edge_index[1].reshape(_NW, 2, _CHALF, _CH)
    zeros = jnp.zeros((_NPAD, _HID), jnp.float32)
    h = _tc_proj(z, W_proj, b_proj)
    for d in range(_DEPTH):
        m2 = _sc_segment_sum(h, src_w, dst_w, zeros)
        if d < _DEPTH - 1:
            h = _tc_gru(m2, h, Wi, Wh, bi, bh)
    return _tc_gru(m2, h, Wi, Wh, bi, bh, head=(W_out, b_out))


# speedup vs baseline: 1.3575x; 1.3575x over previous
import jax
import jax.numpy as jnp
from jax import lax
from jax.experimental import pallas as pl
from jax.experimental.pallas import tpu as pltpu
from jax.experimental.pallas import tpu_sc as plsc

_N = 10000
_E = 320000
_EMB = 128
_HID = 128
_DEPTH = 3
_LAYERS = 2

_NC = 2
_NS = 16
_NW = _NC * _NS
_EPW = _E // _NW
_CH = 80
_NCHUNK = _EPW // _CH
_NPAD = 10240
_RPT = _NPAD // _NS

_BLK = 1000


def _sc_segment_sum(h, src_w, dst_w, zeros):
    mesh = plsc.VectorSubcoreMesh(core_axis_name="c", subcore_axis_name="s")

    def body(h_hbm, src_hbm, dst_hbm, zero_hbm, out_hbm,
             src_v, dst_v, rows_v, m_sh, sem):
        c = lax.axis_index("c")
        s = lax.axis_index("s")
        wid = c * _NS + s
        pltpu.sync_copy(zero_hbm.at[pl.ds(s * _RPT, _RPT)],
                        m_sh.at[pl.ds(s * _RPT, _RPT)])
        pltpu.sync_copy(src_hbm.at[wid], src_v)
        pltpu.sync_copy(dst_hbm.at[wid], dst_v)
        plsc.subcore_barrier()

        def chunk(j, carry):
            pltpu.async_copy(h_hbm.at[src_v.at[j]], rows_v, sem).wait()
            pltpu.sync_copy(rows_v, m_sh.at[dst_v.at[j]], add=True)
            return carry

        lax.fori_loop(0, _NCHUNK, chunk, 0)
        plsc.subcore_barrier()
        pltpu.sync_copy(m_sh.at[pl.ds(s * _RPT, _RPT)],
                        out_hbm.at[c, pl.ds(s * _RPT, _RPT)])

    f = pl.kernel(
        body,
        out_type=jax.ShapeDtypeStruct((_NC, _NPAD, _HID), jnp.float32),
        mesh=mesh,
        scratch_types=[
            pltpu.VMEM((_NCHUNK, _CH), jnp.int32),
            pltpu.VMEM((_NCHUNK, _CH), jnp.int32),
            pltpu.VMEM((_CH, _HID), jnp.float32),
            pltpu.VMEM_SHARED((_NPAD, _HID), jnp.float32),
            pltpu.SemaphoreType.DMA,
        ],
    )
    return f(h, src_w, dst_w, zeros)


def _tc_proj(z, W_proj, b_proj):
    def body(z_ref, w_ref, b_ref, o_ref):
        o_ref[...] = (jnp.dot(z_ref[...], w_ref[...],
                              preferred_element_type=jnp.float32)
                      + b_ref[...])

    return pl.pallas_call(
        body,
        grid=(_N // _BLK,),
        in_specs=[
            pl.BlockSpec((_BLK, _EMB), lambda i: (i, 0)),
            pl.BlockSpec((_EMB, _HID), lambda i: (0, 0)),
            pl.BlockSpec((1, _HID), lambda i: (0, 0)),
        ],
        out_specs=pl.BlockSpec((_BLK, _HID), lambda i: (i, 0)),
        out_shape=jax.ShapeDtypeStruct((_N, _HID), jnp.float32),
    )(z, W_proj, b_proj.reshape(1, _HID))


def _tc_gru(m2, h, Wi, Wh, bi, bh, head=None):
    def body(m_ref, h_ref, wi_ref, wh_ref, bi_ref, bh_ref, *rest):
        inp = m_ref[0] + m_ref[1]
        hcur = h_ref[...]
        for l in range(_LAYERS):
            gi = (jnp.dot(inp, wi_ref[l], preferred_element_type=jnp.float32)
                  + bi_ref[l])
            gh = (jnp.dot(hcur, wh_ref[l], preferred_element_type=jnp.float32)
                  + bh_ref[l])
            r = jax.nn.sigmoid(gi[:, :_HID] + gh[:, :_HID])
            zg = jax.nn.sigmoid(gi[:, _HID:2 * _HID] + gh[:, _HID:2 * _HID])
            n = jnp.tanh(gi[:, 2 * _HID:] + r * gh[:, 2 * _HID:])
            hcur = (1.0 - zg) * n + zg * hcur
            inp = hcur
        if head is None:
            rest[-1][...] = hcur
        else:
            wo_ref, bo_ref, o_ref = rest
            o_ref[...] = (jnp.dot(jnp.tanh(hcur), wo_ref[...],
                                  preferred_element_type=jnp.float32)
                          + bo_ref[...])

    in_specs = [
        pl.BlockSpec((_NC, _BLK, _HID), lambda i: (0, i, 0)),
        pl.BlockSpec((_BLK, _HID), lambda i: (i, 0)),
        pl.BlockSpec((_LAYERS, _HID, 3 * _HID), lambda i: (0, 0, 0)),
        pl.BlockSpec((_LAYERS, _HID, 3 * _HID), lambda i: (0, 0, 0)),
        pl.BlockSpec((_LAYERS, 3 * _HID), lambda i: (0, 0)),
        pl.BlockSpec((_LAYERS, 3 * _HID), lambda i: (0, 0)),
    ]
    args = [m2, h, Wi, Wh, bi, bh]
    if head is None:
        out_specs = pl.BlockSpec((_BLK, _HID), lambda i: (i, 0))
        out_shape = jax.ShapeDtypeStruct((_N, _HID), jnp.float32)
    else:
        W_out, b_out = head
        in_specs += [
            pl.BlockSpec((_HID, 1), lambda i: (0, 0)),
            pl.BlockSpec((1, 1), lambda i: (0, 0)),
        ]
        args += [W_out, b_out.reshape(1, 1)]
        out_specs = pl.BlockSpec((_BLK, 1), lambda i: (i, 0))
        out_shape = jax.ShapeDtypeStruct((_N, 1), jnp.float32)

    return pl.pallas_call(
        body,
        grid=(_N // _BLK,),
        in_specs=in_specs,
        out_specs=out_specs,
        out_shape=out_shape,
    )(*args)


def kernel(z, edge_index, W_proj, b_proj, Wi, Wh, bi, bh, W_out, b_out):
    src_w = edge_index[0].reshape(_NW, _NCHUNK, _CH)
    dst_w = edge_index[1].reshape(_NW, _NCHUNK, _CH)
    zeros = jnp.zeros((_NPAD, _HID), jnp.float32)
    h = _tc_proj(z, W_proj, b_proj)
    for d in range(_DEPTH):
        m2 = _sc_segment_sum(h, src_w, dst_w, zeros)
        if d < _DEPTH - 1:
            h = _tc_gru(m2, h, Wi, Wh, bi, bh)
    return _tc_gru(m2, h, Wi, Wh, bi, bh, head=(W_out, b_out))


# speedup vs baseline: 1.3763x; 1.0139x over previous
import jax
import jax.numpy as jnp
from jax import lax
from jax.experimental import pallas as pl
from jax.experimental.pallas import tpu as pltpu
from jax.experimental.pallas import tpu_sc as plsc

_N = 10000
_E = 320000
_EMB = 128
_HID = 128
_DEPTH = 3
_LAYERS = 2

_NC = 2
_NS = 16
_NW = _NC * _NS
_EPW = _E // _NW
_CH = 80
_NCHUNK = _EPW // _CH
_NPAD = 10240
_RPT = _NPAD // _NS

_BLK = 2000


def _sc_segment_sum(h, src_w, dst_w, zeros):
    mesh = plsc.VectorSubcoreMesh(core_axis_name="c", subcore_axis_name="s")

    def body(h_hbm, src_hbm, dst_hbm, zero_hbm, out_hbm,
             src_v, dst_v, rows_v, m_sh, sem):
        c = lax.axis_index("c")
        s = lax.axis_index("s")
        wid = c * _NS + s
        pltpu.sync_copy(zero_hbm.at[pl.ds(s * _RPT, _RPT)],
                        m_sh.at[pl.ds(s * _RPT, _RPT)])
        pltpu.sync_copy(src_hbm.at[wid], src_v)
        pltpu.sync_copy(dst_hbm.at[wid], dst_v)
        plsc.subcore_barrier()

        def chunk(j, carry):
            pltpu.async_copy(h_hbm.at[src_v.at[j]], rows_v, sem).wait()
            pltpu.sync_copy(rows_v, m_sh.at[dst_v.at[j]], add=True)
            return carry

        lax.fori_loop(0, _NCHUNK, chunk, 0)
        plsc.subcore_barrier()
        pltpu.sync_copy(m_sh.at[pl.ds(s * _RPT, _RPT)],
                        out_hbm.at[c, pl.ds(s * _RPT, _RPT)])

    f = pl.kernel(
        body,
        out_type=jax.ShapeDtypeStruct((_NC, _NPAD, _HID), jnp.float32),
        mesh=mesh,
        scratch_types=[
            pltpu.VMEM((_NCHUNK, _CH), jnp.int32),
            pltpu.VMEM((_NCHUNK, _CH), jnp.int32),
            pltpu.VMEM((_CH, _HID), jnp.float32),
            pltpu.VMEM_SHARED((_NPAD, _HID), jnp.float32),
            pltpu.SemaphoreType.DMA,
        ],
    )
    return f(h, src_w, dst_w, zeros)


def _tc_proj(z, W_proj, b_proj):
    def body(z_ref, w_ref, b_ref, o_ref):
        o_ref[...] = (jnp.dot(z_ref[...], w_ref[...],
                              preferred_element_type=jnp.float32)
                      + b_ref[...])

    return pl.pallas_call(
        body,
        grid=(_N // _BLK,),
        in_specs=[
            pl.BlockSpec((_BLK, _EMB), lambda i: (i, 0)),
            pl.BlockSpec((_EMB, _HID), lambda i: (0, 0)),
            pl.BlockSpec((1, _HID), lambda i: (0, 0)),
        ],
        out_specs=pl.BlockSpec((_BLK, _HID), lambda i: (i, 0)),
        out_shape=jax.ShapeDtypeStruct((_N, _HID), jnp.float32),
    )(z, W_proj, b_proj.reshape(1, _HID))


def _tc_gru(m2, h, Wi, Wh, bi, bh, head=None):
    def body(m_ref, h_ref, wi_ref, wh_ref, bi_ref, bh_ref, *rest):
        inp = m_ref[0] + m_ref[1]
        hcur = h_ref[...]
        for l in range(_LAYERS):
            gi = (jnp.dot(inp, wi_ref[l], preferred_element_type=jnp.float32)
                  + bi_ref[l])
            gh = (jnp.dot(hcur, wh_ref[l], preferred_element_type=jnp.float32)
                  + bh_ref[l])
            r = jax.nn.sigmoid(gi[:, :_HID] + gh[:, :_HID])
            zg = jax.nn.sigmoid(gi[:, _HID:2 * _HID] + gh[:, _HID:2 * _HID])
            n = jnp.tanh(gi[:, 2 * _HID:] + r * gh[:, 2 * _HID:])
            hcur = (1.0 - zg) * n + zg * hcur
            inp = hcur
        if head is None:
            rest[-1][...] = hcur
        else:
            wo_ref, bo_ref, o_ref = rest
            o_ref[...] = (jnp.dot(jnp.tanh(hcur), wo_ref[...],
                                  preferred_element_type=jnp.float32)
                          + bo_ref[...])

    in_specs = [
        pl.BlockSpec((_NC, _BLK, _HID), lambda i: (0, i, 0)),
        pl.BlockSpec((_BLK, _HID), lambda i: (i, 0)),
        pl.BlockSpec((_LAYERS, _HID, 3 * _HID), lambda i: (0, 0, 0)),
        pl.BlockSpec((_LAYERS, _HID, 3 * _HID), lambda i: (0, 0, 0)),
        pl.BlockSpec((_LAYERS, 3 * _HID), lambda i: (0, 0)),
        pl.BlockSpec((_LAYERS, 3 * _HID), lambda i: (0, 0)),
    ]
    args = [m2, h, Wi, Wh, bi, bh]
    if head is None:
        out_specs = pl.BlockSpec((_BLK, _HID), lambda i: (i, 0))
        out_shape = jax.ShapeDtypeStruct((_N, _HID), jnp.float32)
    else:
        W_out, b_out = head
        in_specs += [
            pl.BlockSpec((_HID, 1), lambda i: (0, 0)),
            pl.BlockSpec((1, 1), lambda i: (0, 0)),
        ]
        args += [W_out, b_out.reshape(1, 1)]
        out_specs = pl.BlockSpec((_BLK, 1), lambda i: (i, 0))
        out_shape = jax.ShapeDtypeStruct((_N, 1), jnp.float32)

    return pl.pallas_call(
        body,
        grid=(_N // _BLK,),
        in_specs=in_specs,
        out_specs=out_specs,
        out_shape=out_shape,
    )(*args)


def kernel(z, edge_index, W_proj, b_proj, Wi, Wh, bi, bh, W_out, b_out):
    src_w = edge_index[0].reshape(_NW, _NCHUNK, _CH)
    dst_w = edge_index[1].reshape(_NW, _NCHUNK, _CH)
    zeros = jnp.zeros((_NPAD, _HID), jnp.float32)
    h = _tc_proj(z, W_proj, b_proj)
    for d in range(_DEPTH):
        m2 = _sc_segment_sum(h, src_w, dst_w, zeros)
        if d < _DEPTH - 1:
            h = _tc_gru(m2, h, Wi, Wh, bi, bh)
    return _tc_gru(m2, h, Wi, Wh, bi, bh, head=(W_out, b_out))
